# baseline (speedup 1.0000x reference)
import functools

import jax
import jax.numpy as jnp
from jax.experimental import pallas as pl
from jax.experimental.pallas import tpu as pltpu

B = 2048
D = 1024
E = 8
A = 16
H = 64
HC = 2 * H
OC = 32
BM = 256


def _dense_body(obs_ref, idx_ref, w1_ref, b1_ref, w2_ref, b2_ref, out_ref):
    x = obs_ref[...]
    idx = idx_ref[...]
    acc = jnp.zeros((BM, OC), jnp.float32)
    for e in range(E):
        h = jnp.tanh(
            jax.lax.dot_general(
                x, w1_ref[e],
                (((1,), (0,)), ((), ())),
                preferred_element_type=jnp.float32,
            )
            + b1_ref[e][None, :]
        )
        o = (
            jax.lax.dot_general(
                h, w2_ref[e],
                (((1,), (0,)), ((), ())),
                preferred_element_type=jnp.float32,
            )
            + b2_ref[e][None, :]
        )
        acc = jnp.where(idx == e, o, acc)
    out_ref[...] = acc


@jax.jit
def kernel(obs, idxs, Wa1, ba1, Wa2, ba2, Wc1, bc1, Wc2, bc2):
    w1 = jnp.concatenate([Wa1, Wc1], axis=2)
    b1 = jnp.concatenate([ba1, bc1], axis=1)
    w2 = jnp.zeros((E, HC, OC), jnp.float32)
    w2 = w2.at[:, :H, :A].set(Wa2)
    w2 = w2.at[:, H:, A].set(Wc2[:, :, 0])
    b2 = jnp.zeros((E, OC), jnp.float32)
    b2 = b2.at[:, :A].set(ba2)
    b2 = b2.at[:, A].set(bc2[:, 0])

    idx2 = idxs.astype(jnp.int32).reshape(B, 1)

    out = pl.pallas_call(
        _dense_body,
        grid=(B // BM,),
        in_specs=[
            pl.BlockSpec((BM, D), lambda i: (i, 0)),
            pl.BlockSpec((BM, 1), lambda i: (i, 0)),
            pl.BlockSpec((E, D, HC), lambda i: (0, 0, 0)),
            pl.BlockSpec((E, HC), lambda i: (0, 0)),
            pl.BlockSpec((E, HC, OC), lambda i: (0, 0, 0)),
            pl.BlockSpec((E, OC), lambda i: (0, 0)),
        ],
        out_specs=pl.BlockSpec((BM, OC), lambda i: (i, 0)),
        out_shape=jax.ShapeDtypeStruct((B, OC), jnp.float32),
    )(obs, idx2, w1, b1, w2, b2)

    logits = out[:, :A]
    state_value = out[:, A]
    return (logits, state_value)
